# baseline (speedup 1.0000x reference)
import functools

import numpy as np
import jax
import jax.numpy as jnp
from jax.experimental import pallas as pl
from jax.experimental.pallas import tpu as pltpu

D = 64
S = 16
K = 1000
T = 8
RMAX = 10.0
NG = 8
NUM_LAYERS = 2

BE = 2048

_C1 = float(np.sqrt(3.0))
_C2 = float(np.sqrt(15.0))
_C3 = float(np.sqrt(5.0) / 2.0)
_BESSEL_C = float(np.sqrt(2.0 / RMAX))


def _edge_msg_body(vec_ref, h2s_ref, val_ref, w1_ref, b1_ref, w2_ref,
                   b2_ref, wsh_ref, out_ref):
    vec = vec_ref[...]
    x = vec[:, 0:1]
    y = vec[:, 1:2]
    z = vec[:, 2:3]
    r = jnp.sqrt(x * x + y * y + z * z)
    inv = 1.0 / (r + 1e-9)
    ux = x * inv
    uy = y * inv
    uz = z * inv
    sh = jnp.concatenate([
        jnp.ones_like(ux), _C1 * uy, _C1 * uz, _C1 * ux,
        _C2 * ux * uy, _C2 * uy * uz, _C3 * (3.0 * uz * uz - 1.0),
        _C2 * ux * uz, (_C2 / 2.0) * (ux * ux - uy * uy)
    ], axis=1)
    nvec = (jax.lax.broadcasted_iota(jnp.int32, (1, 8), 1) + 1
            ).astype(jnp.float32)
    bessel = _BESSEL_C * jnp.sin(nvec * (np.pi / RMAX) * r) * inv
    u = r * (1.0 / RMAX)
    p = 5.0
    u2 = u * u
    u4 = u2 * u2
    u5 = u4 * u
    u6 = u5 * u
    u7 = u6 * u
    env = 1.0 - ((p + 1.0) * (p + 2.0) / 2.0) * u5 \
        + p * (p + 2.0) * u6 - (p * (p + 1.0) / 2.0) * u7
    env = jnp.where(u < 1.0, env, 0.0)
    ef = bessel * env
    hidden = jnp.maximum(
        jnp.dot(ef, w1_ref[...], preferred_element_type=jnp.float32)
        + b1_ref[...], 0.0)
    w = jnp.dot(hidden, w2_ref[...], preferred_element_type=jnp.float32) \
        + b2_ref[...]
    sv = jnp.dot(sh, wsh_ref[...], preferred_element_type=jnp.float32)
    out_ref[...] = (h2s_ref[...] * w + sv) * val_ref[...]


def _edge_msg(vecs, h2src, valid, p):
    ep = vecs.shape[0]
    grid = ep // BE
    return pl.pallas_call(
        _edge_msg_body,
        grid=(grid,),
        in_specs=[
            pl.BlockSpec((BE, 3), lambda i: (i, 0)),
            pl.BlockSpec((BE, D), lambda i: (i, 0)),
            pl.BlockSpec((BE, 1), lambda i: (i, 0)),
            pl.BlockSpec((8, 256), lambda i: (0, 0)),
            pl.BlockSpec((1, 256), lambda i: (0, 0)),
            pl.BlockSpec((256, D), lambda i: (0, 0)),
            pl.BlockSpec((1, D), lambda i: (0, 0)),
            pl.BlockSpec((9, D), lambda i: (0, 0)),
        ],
        out_specs=pl.BlockSpec((BE, D), lambda i: (i, 0)),
        out_shape=jax.ShapeDtypeStruct((ep, D), jnp.float32),
    )(vecs, h2src, valid,
      p['W_r1'], p['b_r1'].reshape(1, 256),
      p['W_r2'], p['b_r2'].reshape(1, D),
      p['W_sh'])


def _pad_edges(src, dst, valid):
    e = src.shape[0]
    ep = ((e + BE - 1) // BE) * BE
    pad = ep - e
    if pad:
        src = jnp.concatenate([src, jnp.zeros((pad,), jnp.int32)])
        dst = jnp.concatenate([dst, jnp.zeros((pad,), jnp.int32)])
        valid = jnp.concatenate([valid, jnp.zeros((pad,), jnp.float32)])
    return src, dst, valid


def _conv(h_tab, pos_tab, src, dst, valid, num_nodes, p):
    h2 = h_tab @ p['W_msg']
    h2src = jnp.take(h2, src, axis=0)
    vecs = jnp.take(pos_tab, src, axis=0) - jnp.take(pos_tab, dst, axis=0)
    msg = _edge_msg(vecs, h2src, valid[:, None], p)
    agg = jax.ops.segment_sum(msg, dst, num_segments=num_nodes)
    return agg * jax.nn.sigmoid(agg @ p['W_gate'] + p['b_gate'])


def _hmp_layer(h, pos, src, dst, ones_pad, p):
    n = h.shape[0]
    h_update = _conv(h, pos, src, dst, ones_pad, n, p)
    h_local = h_update + h
    h_scalar = h_local[:, :S]
    score = (jax.nn.relu(h_scalar @ p['W_ms1'] + p['b_ms1']) @ p['W_ms2']
             + p['b_ms2'])[:, 0]
    m = jax.nn.sigmoid(score)
    _, master_idx = jax.lax.top_k(score, K)
    rank = jnp.full((n,), K, dtype=jnp.int32).at[master_idx].set(
        jnp.arange(K, dtype=jnp.int32))
    e = src.shape[0]
    sr = jnp.take(rank, src, axis=0)
    dr = jnp.take(rank, dst, axis=0)
    valid_ind = (sr < K) & (dr < K) & (ones_pad > 0)
    h_master = jnp.take(h_local, master_idx, axis=0)
    pos_master = jnp.take(pos, master_idx, axis=0)
    hs = h_master[:, :S]
    logits = (hs @ p['Wq']) @ (hs @ p['Wk']).T / np.sqrt(S)
    attn = jax.nn.softmax(logits, axis=-1)
    _, vcols = jax.lax.top_k(attn, T)
    v_src = jnp.repeat(jnp.arange(K, dtype=jnp.int32), T)
    v_dst = vcols.reshape(-1).astype(jnp.int32)
    v_ok = (v_src != v_dst).astype(jnp.float32)
    e_src = jnp.concatenate([jnp.where(valid_ind, sr, K), v_src, v_dst])
    e_dst = jnp.concatenate([jnp.where(valid_ind, dr, K), v_dst, v_src])
    e_valid = jnp.concatenate([valid_ind.astype(jnp.float32), v_ok, v_ok])
    e_src, e_dst, e_valid = _pad_edges(e_src, e_dst, e_valid)
    hm_pad = jnp.concatenate(
        [h_master, jnp.zeros((1, D), dtype=h_master.dtype)], axis=0)
    pm_pad = jnp.concatenate(
        [pos_master, jnp.zeros((1, 3), dtype=pos_master.dtype)], axis=0)
    hm_update = _conv(hm_pad, pm_pad, e_src, e_dst, e_valid, K + 1, p)[:K]
    h_hier = hm_update + h_master
    h_hier_exp = jnp.zeros_like(h_local).at[master_idx].set(h_hier)
    m_exp = m[:, None]
    return (1.0 - m_exp) * h_local + m_exp * h_hier_exp


def kernel(atoms, pos, edge_index, batch, params):
    h = params['emb'][atoms]
    src, dst = edge_index[0], edge_index[1]
    e = src.shape[0]
    ones = jnp.ones((e,), dtype=jnp.float32)
    src_p, dst_p, ones_p = _pad_edges(src, dst, ones)
    for i in range(NUM_LAYERS):
        h = _hmp_layer(h, pos, src_p, dst_p, ones_p, params['layers'][i])
    pooled = jax.ops.segment_sum(h[:, :D], batch, num_segments=NG)
    hidden = jax.nn.relu(pooled @ params['W_p1'] + params['b_p1'])
    return hidden @ params['W_p2'] + params['b_p2']


# speedup vs baseline: 1.4113x; 1.4113x over previous
import functools

import numpy as np
import jax
from jax import lax
import jax.numpy as jnp
from jax.experimental import pallas as pl
from jax.experimental.pallas import tpu as pltpu
from jax.experimental.pallas import tpu_sc as plsc

D = 64
S = 16
K = 1000
T = 8
RMAX = 10.0
NG = 8
NUM_LAYERS = 2

BE = 2048
NW = 32
CHI = 128
EP_ALIGN = 32768

_C1 = float(np.sqrt(3.0))
_C2 = float(np.sqrt(15.0))
_C3 = float(np.sqrt(5.0) / 2.0)
_BESSEL_C = float(np.sqrt(2.0 / RMAX))

@functools.lru_cache(maxsize=None)
def _mesh():
    return plsc.VectorSubcoreMesh(core_axis_name="c", subcore_axis_name="s")


def _wid():
    return lax.axis_index("s") * 2 + lax.axis_index("c")


@functools.lru_cache(maxsize=None)
def _make_gather2(ep, n1, d1, n2, d2):
    rows_w = ep // NW
    cb = 512
    nbc = rows_w // cb
    grp = cb // CHI
    nch = rows_w // CHI

    @functools.partial(
        pl.kernel, mesh=_mesh(),
        compiler_params=pltpu.CompilerParams(use_tc_tiling_on_sc=False, needs_layout_passes=False),
        out_type=[jax.ShapeDtypeStruct((ep, d1), jnp.float32),
                  jax.ShapeDtypeStruct((ep, d2), jnp.float32)],
        scratch_types=[pltpu.VMEM((grp, CHI), jnp.int32),
                       pltpu.VMEM((grp, CHI), jnp.int32),
                       pltpu.VMEM((cb, d1), jnp.float32),
                       pltpu.VMEM((cb, d2), jnp.float32),
                       pltpu.SemaphoreType.DMA],
    )
    def kfn(tab1, idx1, tab2, idx2, out1, out2, i1v, i2v, r1v, r2v, sem):
        wid = _wid()

        def chunk(j, carry):
            base = wid * rows_w + j * cb
            pltpu.sync_copy(idx1.at[pl.ds(wid * nch + j * grp, grp)], i1v)
            pltpu.sync_copy(idx2.at[pl.ds(wid * nch + j * grp, grp)], i2v)
            cps = []
            for k in range(grp):
                cps.append(pltpu.async_copy(
                    tab1.at[i1v.at[k]],
                    r1v.at[pl.ds(k * CHI, CHI)], sem))
                cps.append(pltpu.async_copy(
                    tab2.at[i2v.at[k]],
                    r2v.at[pl.ds(k * CHI, CHI)], sem))
            for c in cps:
                c.wait()
            pltpu.sync_copy(r1v, out1.at[pl.ds(base, cb)])
            pltpu.sync_copy(r2v, out2.at[pl.ds(base, cb)])
            return carry

        lax.fori_loop(0, nbc, chunk, 0)

    return kfn


@functools.lru_cache(maxsize=None)
def _make_gather1(ep, n1, d1, chi=CHI):
    rows_w = ep // NW
    cb = min(512, rows_w)
    nbc = rows_w // cb
    grp = cb // chi
    nch = rows_w // chi

    @functools.partial(
        pl.kernel, mesh=_mesh(),
        compiler_params=pltpu.CompilerParams(use_tc_tiling_on_sc=False, needs_layout_passes=False),
        out_type=jax.ShapeDtypeStruct((ep, d1), jnp.float32),
        scratch_types=[pltpu.VMEM((grp, chi), jnp.int32),
                       pltpu.VMEM((cb, d1), jnp.float32),
                       pltpu.SemaphoreType.DMA],
    )
    def kfn(tab1, idx1, out1, i1v, r1v, sem):
        wid = _wid()

        def chunk(j, carry):
            base = wid * rows_w + j * cb
            pltpu.sync_copy(idx1.at[pl.ds(wid * nch + j * grp, grp)], i1v)
            cps = []
            for k in range(grp):
                cps.append(pltpu.async_copy(
                    tab1.at[i1v.at[k]],
                    r1v.at[pl.ds(k * chi, chi)], sem))
            for c in cps:
                c.wait()
            pltpu.sync_copy(r1v, out1.at[pl.ds(base, cb)])
            return carry

        lax.fori_loop(0, nbc, chunk, 0)

    return kfn


def _sc_gather_rows(tab, idx, ep, chi=CHI):
    idx_p = _pad1(idx.astype(jnp.int32), ep, 0)
    return _make_gather1(ep, tab.shape[0], tab.shape[1], chi)(
        tab, idx_p.reshape(ep // chi, chi))


@functools.lru_cache(maxsize=None)
def _make_scatter(ep, npad):
    rows_w = ep // NW
    cb = 512
    nbc = rows_w // cb
    grp = cb // CHI
    nch = rows_w // CHI
    rows_t = npad // 16
    nsp = 2 if rows_t > 128 else 1
    tb = rows_t // nsp

    @functools.partial(
        pl.kernel, mesh=_mesh(),
        compiler_params=pltpu.CompilerParams(use_tc_tiling_on_sc=False, needs_layout_passes=False),
        out_type=jax.ShapeDtypeStruct((2, npad, D), jnp.float32),
        scratch_types=[pltpu.VMEM_SHARED((npad, D), jnp.float32),
                       pltpu.VMEM((grp, CHI), jnp.int32),
                       pltpu.VMEM((cb, D), jnp.float32),
                       pltpu.VMEM((tb, D), jnp.float32)],
    )
    def kfn(msg, dstg, zeros, out, acc, idxv, msgv, tbuf):
        cid = lax.axis_index("c")
        sid = lax.axis_index("s")
        wid = sid * 2 + cid
        for q in range(nsp):
            pltpu.sync_copy(zeros.at[pl.ds(sid * rows_t + q * tb, tb)], tbuf)
            pltpu.sync_copy(tbuf, acc.at[pl.ds(sid * rows_t + q * tb, tb)])
        plsc.subcore_barrier()

        def chunk(j, carry):
            base = wid * rows_w + j * cb
            pltpu.sync_copy(dstg.at[pl.ds(wid * nch + j * grp, grp)], idxv)
            pltpu.sync_copy(msg.at[pl.ds(base, cb)], msgv)
            for k in range(grp):
                pltpu.sync_copy(msgv.at[pl.ds(k * CHI, CHI)],
                                acc.at[idxv.at[k]], add=True)
            return carry

        lax.fori_loop(0, nbc, chunk, 0)
        plsc.subcore_barrier()
        for q in range(nsp):
            pltpu.sync_copy(acc.at[pl.ds(sid * rows_t + q * tb, tb)], tbuf)
            pltpu.sync_copy(
                tbuf, out.at[cid].at[pl.ds(sid * rows_t + q * tb, tb)])

    return kfn


@functools.lru_cache(maxsize=None)
def _make_rankmap(ep, npr, e_real):
    rows_w = ep // NW
    cbe = 2048
    nbc = rows_w // cbe

    @functools.partial(
        pl.kernel, mesh=_mesh(),
        compiler_params=pltpu.CompilerParams(use_tc_tiling_on_sc=False, needs_layout_passes=False),
        out_type=[jax.ShapeDtypeStruct((ep,), jnp.int32),
                  jax.ShapeDtypeStruct((ep,), jnp.int32),
                  jax.ShapeDtypeStruct((ep,), jnp.float32),
                  jax.ShapeDtypeStruct((npr,), jnp.int32)],
        scratch_types=[pltpu.VMEM((npr,), jnp.int32),
                       pltpu.VMEM((1024,), jnp.int32),
                       pltpu.VMEM((cbe,), jnp.int32),
                       pltpu.VMEM((cbe,), jnp.int32),
                       pltpu.VMEM((cbe,), jnp.int32),
                       pltpu.VMEM((cbe,), jnp.int32),
                       pltpu.VMEM((cbe,), jnp.float32)],
    )
    def kfn(mi_hbm, src_hbm, dst_hbm, o_src, o_dst, o_val, o_rank,
            rank_v, mi_v, sbuf, dbuf, eob, dob, vbuf):
        wid = _wid()
        pltpu.sync_copy(mi_hbm, mi_v)
        ksplat = jnp.full((16,), K, jnp.int32)

        def initf(i, carry):
            rank_v[pl.ds(i * 16, 16)] = ksplat
            return carry

        lax.fori_loop(0, npr // 16, initf, 0)

        def setf(j, carry):
            idx16 = mi_v[pl.ds(j * 16, 16)]
            val16 = j * 16 + lax.iota(jnp.int32, 16)
            plsc.store_scatter(rank_v, [idx16], val16, mask=val16 < K)
            return carry

        lax.fori_loop(0, 1024 // 16, setf, 0)

        @pl.when(wid == 0)
        def _():
            pltpu.sync_copy(rank_v, o_rank)

        def chunk(j, carry):
            base = wid * rows_w + j * cbe
            pltpu.sync_copy(src_hbm.at[pl.ds(base, cbe)], sbuf)
            pltpu.sync_copy(dst_hbm.at[pl.ds(base, cbe)], dbuf)

            def grpf(g, c2):
                s16 = sbuf[pl.ds(g * 16, 16)]
                d16 = dbuf[pl.ds(g * 16, 16)]
                sr = plsc.load_gather(rank_v, [s16])
                dr = plsc.load_gather(rank_v, [d16])
                gpos = base + g * 16 + lax.iota(jnp.int32, 16)
                ok = (sr < K) & (dr < K) & (gpos < e_real)
                eob[pl.ds(g * 16, 16)] = jnp.where(ok, sr, K)
                dob[pl.ds(g * 16, 16)] = jnp.where(ok, dr, K)
                vbuf[pl.ds(g * 16, 16)] = jnp.where(
                    ok, jnp.float32(1.0), jnp.float32(0.0))
                return c2

            lax.fori_loop(0, cbe // 16, grpf, 0)
            pltpu.sync_copy(eob, o_src.at[pl.ds(base, cbe)])
            pltpu.sync_copy(dob, o_dst.at[pl.ds(base, cbe)])
            pltpu.sync_copy(vbuf, o_val.at[pl.ds(base, cbe)])
            return carry

        lax.fori_loop(0, nbc, chunk, 0)

    return kfn


def _edge_msg_body(g1_ref, g2_ref, val_ref, w1_ref, b1_ref, w2_ref,
                   b2_ref, wsh_ref, out_ref):
    g1 = g1_ref[...]
    h2s = g1[:, :D]
    x = g1[:, D:D + 1] - g2_ref[:, 0:1]
    y = g1[:, D + 1:D + 2] - g2_ref[:, 1:2]
    z = g1[:, D + 2:D + 3] - g2_ref[:, 2:3]
    r = jnp.sqrt(x * x + y * y + z * z)
    inv = 1.0 / (r + 1e-9)
    ux = x * inv
    uy = y * inv
    uz = z * inv
    sh = jnp.concatenate([
        jnp.ones_like(ux), _C1 * uy, _C1 * uz, _C1 * ux,
        _C2 * ux * uy, _C2 * uy * uz, _C3 * (3.0 * uz * uz - 1.0),
        _C2 * ux * uz, (_C2 / 2.0) * (ux * ux - uy * uy)
    ], axis=1)
    nvec = (lax.broadcasted_iota(jnp.int32, (1, 8), 1) + 1
            ).astype(jnp.float32)
    bessel = _BESSEL_C * jnp.sin(nvec * (np.pi / RMAX) * r) * inv
    u = r * (1.0 / RMAX)
    p = 5.0
    u2 = u * u
    u4 = u2 * u2
    u5 = u4 * u
    u6 = u5 * u
    u7 = u6 * u
    env = 1.0 - ((p + 1.0) * (p + 2.0) / 2.0) * u5 \
        + p * (p + 2.0) * u6 - (p * (p + 1.0) / 2.0) * u7
    env = jnp.where(u < 1.0, env, 0.0)
    ef = bessel * env
    hidden = jnp.maximum(
        jnp.dot(ef, w1_ref[...], preferred_element_type=jnp.float32)
        + b1_ref[...], 0.0)
    w = jnp.dot(hidden, w2_ref[...], preferred_element_type=jnp.float32) \
        + b2_ref[...]
    sv = jnp.dot(sh, wsh_ref[...], preferred_element_type=jnp.float32)
    out_ref[...] = (h2s * w + sv) * val_ref[...]


def _edge_msg(g1, g2, valid, p):
    ep = g1.shape[0]
    grid = ep // BE
    return pl.pallas_call(
        _edge_msg_body,
        grid=(grid,),
        in_specs=[
            pl.BlockSpec((BE, 80), lambda i: (i, 0)),
            pl.BlockSpec((BE, 16), lambda i: (i, 0)),
            pl.BlockSpec((BE, 1), lambda i: (i, 0)),
            pl.BlockSpec((8, 256), lambda i: (0, 0)),
            pl.BlockSpec((1, 256), lambda i: (0, 0)),
            pl.BlockSpec((256, D), lambda i: (0, 0)),
            pl.BlockSpec((1, D), lambda i: (0, 0)),
            pl.BlockSpec((9, D), lambda i: (0, 0)),
        ],
        out_specs=pl.BlockSpec((BE, D), lambda i: (i, 0)),
        out_shape=jax.ShapeDtypeStruct((ep, D), jnp.float32),
    )(g1, g2, valid,
      p['W_r1'], p['b_r1'].reshape(1, 256),
      p['W_r2'], p['b_r2'].reshape(1, D),
      p['W_sh'])


def _pad1(x, ep, fill):
    pad = ep - x.shape[0]
    return jnp.concatenate([x, jnp.full((pad,), fill, x.dtype)]) if pad else x


def _conv(h_tab, pos_tab, src_p, dst_p, valid_p, num_nodes, npad, p):
    ep = src_p.shape[0]
    n = num_nodes
    h2 = h_tab @ p['W_msg']
    zpad = jnp.zeros((n, 13), jnp.float32)
    tab1 = jnp.concatenate([h2, pos_tab, zpad], axis=1)
    tab2 = jnp.concatenate([pos_tab, zpad], axis=1)
    g1, g2 = _make_gather2(ep, n, 80, n, 16)(
        tab1, src_p.reshape(ep // CHI, CHI), tab2,
        dst_p.reshape(ep // CHI, CHI))
    msg = _edge_msg(g1, g2, valid_p[:, None], p)
    parts = _make_scatter(ep, npad)(
        msg, dst_p.reshape(ep // CHI, CHI), jnp.zeros((npad, D), jnp.float32))
    agg = parts[0, :n] + parts[1, :n]
    return agg * jax.nn.sigmoid(agg @ p['W_gate'] + p['b_gate'])


def _hmp_layer(h, pos, src_p, dst_p, ones_p, e_real, p):
    n = h.shape[0]
    ep = src_p.shape[0]
    h_update = _conv(h, pos, src_p, dst_p, ones_p, n, 10240, p)
    h_local = h_update + h
    h_scalar = h_local[:, :S]
    score = (jax.nn.relu(h_scalar @ p['W_ms1'] + p['b_ms1']) @ p['W_ms2']
             + p['b_ms2'])[:, 0]
    m = jax.nn.sigmoid(score)
    _, master_idx = jax.lax.top_k(score, K)
    npr = 10240
    mi_p = _pad1(master_idx.astype(jnp.int32), 1024, 0)
    e_srcE, e_dstE, e_valE, rank = _make_rankmap(ep, npr, e_real)(
        mi_p, src_p, dst_p)
    hp_tab = jnp.concatenate(
        [h_local, pos, jnp.zeros((n, 13), jnp.float32)], axis=1)
    hp_m = _sc_gather_rows(hp_tab, master_idx, 1024, chi=32)
    h_master = hp_m[:K, :D]
    pos_master = hp_m[:K, D:D + 3]
    hs = h_master[:, :S]
    logits = (hs @ p['Wq']) @ (hs @ p['Wk']).T / np.sqrt(S)
    attn = jax.nn.softmax(logits, axis=-1)
    _, vcols = jax.lax.top_k(attn, T)
    v_src = jnp.repeat(jnp.arange(K, dtype=jnp.int32), T)
    v_dst = vcols.reshape(-1).astype(jnp.int32)
    v_ok = (v_src != v_dst).astype(jnp.float32)
    epm = (ep + 2 * K * T + EP_ALIGN - 1) // EP_ALIGN * EP_ALIGN
    e_src = _pad1(jnp.concatenate([e_srcE, v_src, v_dst]), epm, 0)
    e_dst = _pad1(jnp.concatenate([e_dstE, v_dst, v_src]), epm, 0)
    e_val = _pad1(jnp.concatenate([e_valE, v_ok, v_ok]), epm, 0.0)
    hm_pad = jnp.concatenate(
        [h_master, jnp.zeros((1, D), dtype=h_master.dtype)], axis=0)
    pm_pad = jnp.concatenate(
        [pos_master, jnp.zeros((1, 3), dtype=pos_master.dtype)], axis=0)
    hm_update = _conv(hm_pad, pm_pad, e_src, e_dst, e_val, K + 1, 1024, p)[:K]
    h_hier = hm_update + h_master
    hh_tab = jnp.concatenate([h_hier, jnp.zeros((1, D), h_hier.dtype)])
    h_hier_exp = _sc_gather_rows(hh_tab, rank, 10240, chi=64)[:n]
    m_exp = m[:, None]
    return (1.0 - m_exp) * h_local + m_exp * h_hier_exp


def kernel(atoms, pos, edge_index, batch, params):
    emb = params['emb']
    oh_a = (atoms[:, None] == jnp.arange(emb.shape[0], dtype=jnp.int32)
            [None, :]).astype(jnp.float32)
    h = oh_a @ emb
    src, dst = edge_index[0], edge_index[1]
    e = src.shape[0]
    ep = (e + EP_ALIGN - 1) // EP_ALIGN * EP_ALIGN
    src_p = _pad1(src, ep, 0)
    dst_p = _pad1(dst, ep, 0)
    ones_p = _pad1(jnp.ones((e,), jnp.float32), ep, 0.0)
    stacked = jax.tree.map(lambda *xs: jnp.stack(xs), *params['layers'])

    def body(hc, lp):
        return _hmp_layer(hc, pos, src_p, dst_p, ones_p, e, lp), None

    h, _ = jax.lax.scan(body, h, stacked)
    oh_b = (batch[:, None] == jnp.arange(NG, dtype=jnp.int32)
            [None, :]).astype(jnp.float32)
    pooled = oh_b.T @ h[:, :D]
    hidden = jax.nn.relu(pooled @ params['W_p1'] + params['b_p1'])
    return hidden @ params['W_p2'] + params['b_p2']


# speedup vs baseline: 1.4114x; 1.0001x over previous
import functools

import numpy as np
import jax
from jax import lax
import jax.numpy as jnp
from jax.experimental import pallas as pl
from jax.experimental.pallas import tpu as pltpu
from jax.experimental.pallas import tpu_sc as plsc

D = 64
S = 16
K = 1000
T = 8
RMAX = 10.0
NG = 8
NUM_LAYERS = 2

BE = 2048
NW = 32
CHI = 128
EP_ALIGN = 32768

_C1 = float(np.sqrt(3.0))
_C2 = float(np.sqrt(15.0))
_C3 = float(np.sqrt(5.0) / 2.0)
_BESSEL_C = float(np.sqrt(2.0 / RMAX))

@functools.lru_cache(maxsize=None)
def _mesh():
    return plsc.VectorSubcoreMesh(core_axis_name="c", subcore_axis_name="s")


def _wid():
    return lax.axis_index("s") * 2 + lax.axis_index("c")


@functools.lru_cache(maxsize=None)
def _make_gather2(ep, n1, d1, n2, d2):
    rows_w = ep // NW
    cb = 512
    nbc = rows_w // cb
    grp = cb // CHI
    nch = rows_w // CHI

    @functools.partial(
        pl.kernel, mesh=_mesh(),
        compiler_params=pltpu.CompilerParams(use_tc_tiling_on_sc=False, needs_layout_passes=False),
        name=f"sc_gather2_{ep}",
        out_type=[jax.ShapeDtypeStruct((ep, d1), jnp.float32),
                  jax.ShapeDtypeStruct((ep, d2), jnp.float32)],
        scratch_types=[pltpu.VMEM((grp, CHI), jnp.int32),
                       pltpu.VMEM((grp, CHI), jnp.int32),
                       pltpu.VMEM((cb, d1), jnp.float32),
                       pltpu.VMEM((cb, d2), jnp.float32),
                       pltpu.SemaphoreType.DMA],
    )
    def kfn(tab1, idx1, tab2, idx2, out1, out2, i1v, i2v, r1v, r2v, sem):
        wid = _wid()

        def chunk(j, carry):
            base = wid * rows_w + j * cb
            pltpu.sync_copy(idx1.at[pl.ds(wid * nch + j * grp, grp)], i1v)
            pltpu.sync_copy(idx2.at[pl.ds(wid * nch + j * grp, grp)], i2v)
            cps = []
            for k in range(grp):
                cps.append(pltpu.async_copy(
                    tab1.at[i1v.at[k]],
                    r1v.at[pl.ds(k * CHI, CHI)], sem))
                cps.append(pltpu.async_copy(
                    tab2.at[i2v.at[k]],
                    r2v.at[pl.ds(k * CHI, CHI)], sem))
            for c in cps:
                c.wait()
            pltpu.sync_copy(r1v, out1.at[pl.ds(base, cb)])
            pltpu.sync_copy(r2v, out2.at[pl.ds(base, cb)])
            return carry

        lax.fori_loop(0, nbc, chunk, 0)

    return kfn


@functools.lru_cache(maxsize=None)
def _make_gather1(ep, n1, d1, chi=CHI):
    rows_w = ep // NW
    cb = min(512, rows_w)
    nbc = rows_w // cb
    grp = cb // chi
    nch = rows_w // chi

    @functools.partial(
        pl.kernel, mesh=_mesh(),
        compiler_params=pltpu.CompilerParams(use_tc_tiling_on_sc=False, needs_layout_passes=False),
        name=f"sc_gather1_{ep}",
        out_type=jax.ShapeDtypeStruct((ep, d1), jnp.float32),
        scratch_types=[pltpu.VMEM((grp, chi), jnp.int32),
                       pltpu.VMEM((cb, d1), jnp.float32),
                       pltpu.SemaphoreType.DMA],
    )
    def kfn(tab1, idx1, out1, i1v, r1v, sem):
        wid = _wid()

        def chunk(j, carry):
            base = wid * rows_w + j * cb
            pltpu.sync_copy(idx1.at[pl.ds(wid * nch + j * grp, grp)], i1v)
            cps = []
            for k in range(grp):
                cps.append(pltpu.async_copy(
                    tab1.at[i1v.at[k]],
                    r1v.at[pl.ds(k * chi, chi)], sem))
            for c in cps:
                c.wait()
            pltpu.sync_copy(r1v, out1.at[pl.ds(base, cb)])
            return carry

        lax.fori_loop(0, nbc, chunk, 0)

    return kfn


def _sc_gather_rows(tab, idx, ep, chi=CHI):
    idx_p = _pad1(idx.astype(jnp.int32), ep, 0)
    return _make_gather1(ep, tab.shape[0], tab.shape[1], chi)(
        tab, idx_p.reshape(ep // chi, chi))


@functools.lru_cache(maxsize=None)
def _make_scatter(ep, npad):
    rows_w = ep // NW
    cb = 512
    nbc = rows_w // cb
    grp = cb // CHI
    nch = rows_w // CHI
    rows_t = npad // 16
    nsp = 2 if rows_t > 128 else 1
    tb = rows_t // nsp

    @functools.partial(
        pl.kernel, mesh=_mesh(),
        compiler_params=pltpu.CompilerParams(use_tc_tiling_on_sc=False, needs_layout_passes=False),
        name=f"sc_scatter_{npad}",
        out_type=jax.ShapeDtypeStruct((2, npad, D), jnp.float32),
        scratch_types=[pltpu.VMEM_SHARED((npad, D), jnp.float32),
                       pltpu.VMEM((grp, CHI), jnp.int32),
                       pltpu.VMEM((cb, D), jnp.float32),
                       pltpu.VMEM((tb, D), jnp.float32)],
    )
    def kfn(msg, dstg, zeros, out, acc, idxv, msgv, tbuf):
        cid = lax.axis_index("c")
        sid = lax.axis_index("s")
        wid = sid * 2 + cid
        for q in range(nsp):
            pltpu.sync_copy(zeros.at[pl.ds(sid * rows_t + q * tb, tb)], tbuf)
            pltpu.sync_copy(tbuf, acc.at[pl.ds(sid * rows_t + q * tb, tb)])
        plsc.subcore_barrier()

        def chunk(j, carry):
            base = wid * rows_w + j * cb
            pltpu.sync_copy(dstg.at[pl.ds(wid * nch + j * grp, grp)], idxv)
            pltpu.sync_copy(msg.at[pl.ds(base, cb)], msgv)
            for k in range(grp):
                pltpu.sync_copy(msgv.at[pl.ds(k * CHI, CHI)],
                                acc.at[idxv.at[k]], add=True)
            return carry

        lax.fori_loop(0, nbc, chunk, 0)
        plsc.subcore_barrier()
        for q in range(nsp):
            pltpu.sync_copy(acc.at[pl.ds(sid * rows_t + q * tb, tb)], tbuf)
            pltpu.sync_copy(
                tbuf, out.at[cid].at[pl.ds(sid * rows_t + q * tb, tb)])

    return kfn


@functools.lru_cache(maxsize=None)
def _make_rankmap(ep, npr, e_real):
    rows_w = ep // NW
    cbe = 2048
    nbc = rows_w // cbe

    @functools.partial(
        pl.kernel, mesh=_mesh(),
        compiler_params=pltpu.CompilerParams(use_tc_tiling_on_sc=False, needs_layout_passes=False),
        name="sc_rankmap",
        out_type=[jax.ShapeDtypeStruct((ep,), jnp.int32),
                  jax.ShapeDtypeStruct((ep,), jnp.int32),
                  jax.ShapeDtypeStruct((ep,), jnp.float32),
                  jax.ShapeDtypeStruct((npr,), jnp.int32)],
        scratch_types=[pltpu.VMEM((npr,), jnp.int32),
                       pltpu.VMEM((1024,), jnp.int32),
                       pltpu.VMEM((cbe,), jnp.int32),
                       pltpu.VMEM((cbe,), jnp.int32),
                       pltpu.VMEM((cbe,), jnp.int32),
                       pltpu.VMEM((cbe,), jnp.int32),
                       pltpu.VMEM((cbe,), jnp.float32)],
    )
    def kfn(mi_hbm, src_hbm, dst_hbm, o_src, o_dst, o_val, o_rank,
            rank_v, mi_v, sbuf, dbuf, eob, dob, vbuf):
        wid = _wid()
        pltpu.sync_copy(mi_hbm, mi_v)
        ksplat = jnp.full((16,), K, jnp.int32)

        def initf(i, carry):
            rank_v[pl.ds(i * 16, 16)] = ksplat
            return carry

        lax.fori_loop(0, npr // 16, initf, 0)

        def setf(j, carry):
            idx16 = mi_v[pl.ds(j * 16, 16)]
            val16 = j * 16 + lax.iota(jnp.int32, 16)
            plsc.store_scatter(rank_v, [idx16], val16, mask=val16 < K)
            return carry

        lax.fori_loop(0, 1024 // 16, setf, 0)

        @pl.when(wid == 0)
        def _():
            pltpu.sync_copy(rank_v, o_rank)

        def chunk(j, carry):
            base = wid * rows_w + j * cbe
            pltpu.sync_copy(src_hbm.at[pl.ds(base, cbe)], sbuf)
            pltpu.sync_copy(dst_hbm.at[pl.ds(base, cbe)], dbuf)

            def grpf(g, c2):
                s16 = sbuf[pl.ds(g * 16, 16)]
                d16 = dbuf[pl.ds(g * 16, 16)]
                sr = plsc.load_gather(rank_v, [s16])
                dr = plsc.load_gather(rank_v, [d16])
                gpos = base + g * 16 + lax.iota(jnp.int32, 16)
                ok = (sr < K) & (dr < K) & (gpos < e_real)
                eob[pl.ds(g * 16, 16)] = jnp.where(ok, sr, K)
                dob[pl.ds(g * 16, 16)] = jnp.where(ok, dr, K)
                vbuf[pl.ds(g * 16, 16)] = jnp.where(
                    ok, jnp.float32(1.0), jnp.float32(0.0))
                return c2

            lax.fori_loop(0, cbe // 16, grpf, 0)
            pltpu.sync_copy(eob, o_src.at[pl.ds(base, cbe)])
            pltpu.sync_copy(dob, o_dst.at[pl.ds(base, cbe)])
            pltpu.sync_copy(vbuf, o_val.at[pl.ds(base, cbe)])
            return carry

        lax.fori_loop(0, nbc, chunk, 0)

    return kfn


def _edge_msg_body(g1_ref, g2_ref, val_ref, w1_ref, b1_ref, w2_ref,
                   b2_ref, wsh_ref, out_ref):
    g1 = g1_ref[...]
    h2s = g1[:, :D]
    x = g1[:, D:D + 1] - g2_ref[:, 0:1]
    y = g1[:, D + 1:D + 2] - g2_ref[:, 1:2]
    z = g1[:, D + 2:D + 3] - g2_ref[:, 2:3]
    r = jnp.sqrt(x * x + y * y + z * z)
    inv = 1.0 / (r + 1e-9)
    ux = x * inv
    uy = y * inv
    uz = z * inv
    sh = jnp.concatenate([
        jnp.ones_like(ux), _C1 * uy, _C1 * uz, _C1 * ux,
        _C2 * ux * uy, _C2 * uy * uz, _C3 * (3.0 * uz * uz - 1.0),
        _C2 * ux * uz, (_C2 / 2.0) * (ux * ux - uy * uy)
    ], axis=1)
    nvec = (lax.broadcasted_iota(jnp.int32, (1, 8), 1) + 1
            ).astype(jnp.float32)
    bessel = _BESSEL_C * jnp.sin(nvec * (np.pi / RMAX) * r) * inv
    u = r * (1.0 / RMAX)
    p = 5.0
    u2 = u * u
    u4 = u2 * u2
    u5 = u4 * u
    u6 = u5 * u
    u7 = u6 * u
    env = 1.0 - ((p + 1.0) * (p + 2.0) / 2.0) * u5 \
        + p * (p + 2.0) * u6 - (p * (p + 1.0) / 2.0) * u7
    env = jnp.where(u < 1.0, env, 0.0)
    ef = bessel * env
    hidden = jnp.maximum(
        jnp.dot(ef, w1_ref[...], preferred_element_type=jnp.float32)
        + b1_ref[...], 0.0)
    w = jnp.dot(hidden, w2_ref[...], preferred_element_type=jnp.float32) \
        + b2_ref[...]
    sv = jnp.dot(sh, wsh_ref[...], preferred_element_type=jnp.float32)
    out_ref[...] = (h2s * w + sv) * val_ref[...]


def _edge_msg(g1, g2, valid, p):
    ep = g1.shape[0]
    grid = ep // BE
    return pl.pallas_call(
        _edge_msg_body,
        grid=(grid,),
        in_specs=[
            pl.BlockSpec((BE, 80), lambda i: (i, 0)),
            pl.BlockSpec((BE, 16), lambda i: (i, 0)),
            pl.BlockSpec((BE, 1), lambda i: (i, 0)),
            pl.BlockSpec((8, 256), lambda i: (0, 0)),
            pl.BlockSpec((1, 256), lambda i: (0, 0)),
            pl.BlockSpec((256, D), lambda i: (0, 0)),
            pl.BlockSpec((1, D), lambda i: (0, 0)),
            pl.BlockSpec((9, D), lambda i: (0, 0)),
        ],
        out_specs=pl.BlockSpec((BE, D), lambda i: (i, 0)),
        out_shape=jax.ShapeDtypeStruct((ep, D), jnp.float32),
    )(g1, g2, valid,
      p['W_r1'], p['b_r1'].reshape(1, 256),
      p['W_r2'], p['b_r2'].reshape(1, D),
      p['W_sh'])


def _pad1(x, ep, fill):
    pad = ep - x.shape[0]
    return jnp.concatenate([x, jnp.full((pad,), fill, x.dtype)]) if pad else x


def _conv(h_tab, pos_tab, src_p, dst_p, valid_p, num_nodes, npad, p):
    ep = src_p.shape[0]
    n = num_nodes
    h2 = h_tab @ p['W_msg']
    zpad = jnp.zeros((n, 13), jnp.float32)
    tab1 = jnp.concatenate([h2, pos_tab, zpad], axis=1)
    tab2 = jnp.concatenate([pos_tab, zpad], axis=1)
    g1, g2 = _make_gather2(ep, n, 80, n, 16)(
        tab1, src_p.reshape(ep // CHI, CHI), tab2,
        dst_p.reshape(ep // CHI, CHI))
    msg = _edge_msg(g1, g2, valid_p[:, None], p)
    parts = _make_scatter(ep, npad)(
        msg, dst_p.reshape(ep // CHI, CHI), jnp.zeros((npad, D), jnp.float32))
    agg = parts[0, :n] + parts[1, :n]
    return agg * jax.nn.sigmoid(agg @ p['W_gate'] + p['b_gate'])


def _hmp_layer(h, pos, src_p, dst_p, ones_p, e_real, p):
    n = h.shape[0]
    ep = src_p.shape[0]
    h_update = _conv(h, pos, src_p, dst_p, ones_p, n, 10240, p)
    h_local = h_update + h
    h_scalar = h_local[:, :S]
    score = (jax.nn.relu(h_scalar @ p['W_ms1'] + p['b_ms1']) @ p['W_ms2']
             + p['b_ms2'])[:, 0]
    m = jax.nn.sigmoid(score)
    _, master_idx = jax.lax.top_k(score, K)
    npr = 10240
    mi_p = _pad1(master_idx.astype(jnp.int32), 1024, 0)
    e_srcE, e_dstE, e_valE, rank = _make_rankmap(ep, npr, e_real)(
        mi_p, src_p, dst_p)
    hp_tab = jnp.concatenate(
        [h_local, pos, jnp.zeros((n, 13), jnp.float32)], axis=1)
    hp_m = _sc_gather_rows(hp_tab, master_idx, 1024, chi=32)
    h_master = hp_m[:K, :D]
    pos_master = hp_m[:K, D:D + 3]
    hs = h_master[:, :S]
    logits = (hs @ p['Wq']) @ (hs @ p['Wk']).T / np.sqrt(S)
    attn = jax.nn.softmax(logits, axis=-1)
    _, vcols = jax.lax.top_k(attn, T)
    v_src = jnp.repeat(jnp.arange(K, dtype=jnp.int32), T)
    v_dst = vcols.reshape(-1).astype(jnp.int32)
    v_ok = (v_src != v_dst).astype(jnp.float32)
    epm = (ep + 2 * K * T + EP_ALIGN - 1) // EP_ALIGN * EP_ALIGN
    e_src = _pad1(jnp.concatenate([e_srcE, v_src, v_dst]), epm, 0)
    e_dst = _pad1(jnp.concatenate([e_dstE, v_dst, v_src]), epm, 0)
    e_val = _pad1(jnp.concatenate([e_valE, v_ok, v_ok]), epm, 0.0)
    hm_pad = jnp.concatenate(
        [h_master, jnp.zeros((1, D), dtype=h_master.dtype)], axis=0)
    pm_pad = jnp.concatenate(
        [pos_master, jnp.zeros((1, 3), dtype=pos_master.dtype)], axis=0)
    hm_update = _conv(hm_pad, pm_pad, e_src, e_dst, e_val, K + 1, 1024, p)[:K]
    h_hier = hm_update + h_master
    hh_tab = jnp.concatenate([h_hier, jnp.zeros((1, D), h_hier.dtype)])
    h_hier_exp = _sc_gather_rows(hh_tab, rank, 10240, chi=64)[:n]
    m_exp = m[:, None]
    return (1.0 - m_exp) * h_local + m_exp * h_hier_exp


def kernel(atoms, pos, edge_index, batch, params):
    emb = params['emb']
    oh_a = (atoms[:, None] == jnp.arange(emb.shape[0], dtype=jnp.int32)
            [None, :]).astype(jnp.float32)
    h = oh_a @ emb
    src, dst = edge_index[0], edge_index[1]
    e = src.shape[0]
    ep = (e + EP_ALIGN - 1) // EP_ALIGN * EP_ALIGN
    src_p = _pad1(src, ep, 0)
    dst_p = _pad1(dst, ep, 0)
    ones_p = _pad1(jnp.ones((e,), jnp.float32), ep, 0.0)
    stacked = jax.tree.map(lambda *xs: jnp.stack(xs), *params['layers'])

    def body(hc, lp):
        return _hmp_layer(hc, pos, src_p, dst_p, ones_p, e, lp), None

    h, _ = jax.lax.scan(body, h, stacked)
    oh_b = (batch[:, None] == jnp.arange(NG, dtype=jnp.int32)
            [None, :]).astype(jnp.float32)
    pooled = oh_b.T @ h[:, :D]
    hidden = jax.nn.relu(pooled @ params['W_p1'] + params['b_p1'])
    return hidden @ params['W_p2'] + params['b_p2']


# speedup vs baseline: 2.1606x; 1.5308x over previous
import functools

import numpy as np
import jax
from jax import lax
import jax.numpy as jnp
from jax.experimental import pallas as pl
from jax.experimental.pallas import tpu as pltpu
from jax.experimental.pallas import tpu_sc as plsc

D = 64
S = 16
K = 1000
T = 8
RMAX = 10.0
NG = 8
NUM_LAYERS = 2

BE = 2048
NW = 32
CHI = 128
EP_ALIGN = 32768

_C1 = float(np.sqrt(3.0))
_C2 = float(np.sqrt(15.0))
_C3 = float(np.sqrt(5.0) / 2.0)
_BESSEL_C = float(np.sqrt(2.0 / RMAX))

def _ref_sh(vec):
    n = jnp.linalg.norm(vec, axis=-1, keepdims=True)
    u = vec / (n + 1e-9)
    x, y, z = u[..., 0], u[..., 1], u[..., 2]
    return jnp.stack([jnp.ones_like(x), _C1 * y, _C1 * z, _C1 * x,
                      _C2 * x * y, _C2 * y * z,
                      _C3 * (3.0 * z ** 2 - 1.0),
                      _C2 * x * z, (_C2 / 2.0) * (x ** 2 - y ** 2)], axis=-1)


def _ref_radial(r):
    n = jnp.arange(1, 9, dtype=jnp.float32)
    bessel = _BESSEL_C * jnp.sin(n * jnp.pi * r / RMAX) / (r + 1e-9)
    u = r / RMAX
    p = 5.0
    env = 1.0 - ((p + 1.0) * (p + 2.0) / 2.0) * u ** p \
        + p * (p + 2.0) * u ** (p + 1.0) - (p * (p + 1.0) / 2.0) * u ** (p + 2.0)
    env = jnp.where(u < 1.0, env, 0.0)
    return bessel * env


@functools.lru_cache(maxsize=None)
def _mesh():
    return plsc.VectorSubcoreMesh(core_axis_name="c", subcore_axis_name="s")


def _wid():
    return lax.axis_index("s") * 2 + lax.axis_index("c")


@functools.lru_cache(maxsize=None)
def _make_gather2(ep, n1, d1, n2, d2):
    rows_w = ep // NW
    cb = 512
    nbc = rows_w // cb
    grp = cb // CHI
    nch = rows_w // CHI

    @functools.partial(
        pl.kernel, mesh=_mesh(),
        compiler_params=pltpu.CompilerParams(use_tc_tiling_on_sc=False, needs_layout_passes=False),
        name=f"sc_gather2_{ep}",
        out_type=[jax.ShapeDtypeStruct((ep, d1), jnp.float32),
                  jax.ShapeDtypeStruct((ep, d2), jnp.float32)],
        scratch_types=[pltpu.VMEM((grp, CHI), jnp.int32),
                       pltpu.VMEM((grp, CHI), jnp.int32),
                       pltpu.VMEM((cb, d1), jnp.float32),
                       pltpu.VMEM((cb, d2), jnp.float32),
                       pltpu.SemaphoreType.DMA],
    )
    def kfn(tab1, idx1, tab2, idx2, out1, out2, i1v, i2v, r1v, r2v, sem):
        wid = _wid()

        def chunk(j, carry):
            base = wid * rows_w + j * cb
            pltpu.sync_copy(idx1.at[pl.ds(wid * nch + j * grp, grp)], i1v)
            pltpu.sync_copy(idx2.at[pl.ds(wid * nch + j * grp, grp)], i2v)
            cps = []
            for k in range(grp):
                cps.append(pltpu.async_copy(
                    tab1.at[i1v.at[k]],
                    r1v.at[pl.ds(k * CHI, CHI)], sem))
                cps.append(pltpu.async_copy(
                    tab2.at[i2v.at[k]],
                    r2v.at[pl.ds(k * CHI, CHI)], sem))
            for c in cps:
                c.wait()
            pltpu.sync_copy(r1v, out1.at[pl.ds(base, cb)])
            pltpu.sync_copy(r2v, out2.at[pl.ds(base, cb)])
            return carry

        lax.fori_loop(0, nbc, chunk, 0)

    return kfn


@functools.lru_cache(maxsize=None)
def _make_gather2_small(ep, nt, d1, d2):
    rows_w = ep // NW
    cb = 512
    nbc = rows_w // cb
    grp = cb // CHI
    nch = rows_w // CHI

    @functools.partial(
        pl.kernel, mesh=_mesh(),
        compiler_params=pltpu.CompilerParams(
            use_tc_tiling_on_sc=False, needs_layout_passes=False),
        name=f"sc_gather2s_{ep}",
        out_type=[jax.ShapeDtypeStruct((ep, d1), jnp.float32),
                  jax.ShapeDtypeStruct((ep, d2), jnp.float32)],
        scratch_types=[pltpu.VMEM_SHARED((nt, d1), jnp.float32),
                       pltpu.VMEM_SHARED((nt, d2), jnp.float32),
                       pltpu.VMEM((grp, CHI), jnp.int32),
                       pltpu.VMEM((grp, CHI), jnp.int32),
                       pltpu.VMEM((cb, d1), jnp.float32),
                       pltpu.VMEM((cb, d2), jnp.float32),
                       pltpu.SemaphoreType.DMA],
    )
    def kfn(tab1, idx1, tab2, idx2, out1, out2, t1s, t2s, i1v, i2v,
            r1v, r2v, sem):
        sid = lax.axis_index("s")
        cid = lax.axis_index("c")
        wid = sid * 2 + cid

        @pl.when(sid == 0)
        def _():
            for q in range(nt // cb):
                pltpu.sync_copy(tab1.at[pl.ds(q * cb, cb)], r1v)
                pltpu.sync_copy(r1v, t1s.at[pl.ds(q * cb, cb)])
                pltpu.sync_copy(tab2.at[pl.ds(q * cb, cb)], r2v)
                pltpu.sync_copy(r2v, t2s.at[pl.ds(q * cb, cb)])

        plsc.subcore_barrier()

        def chunk(j, carry):
            base = wid * rows_w + j * cb
            pltpu.sync_copy(idx1.at[pl.ds(wid * nch + j * grp, grp)], i1v)
            pltpu.sync_copy(idx2.at[pl.ds(wid * nch + j * grp, grp)], i2v)
            cps = []
            for k in range(grp):
                cps.append(pltpu.async_copy(
                    t1s.at[i1v.at[k]],
                    r1v.at[pl.ds(k * CHI, CHI)], sem))
                cps.append(pltpu.async_copy(
                    t2s.at[i2v.at[k]],
                    r2v.at[pl.ds(k * CHI, CHI)], sem))
            for c in cps:
                c.wait()
            pltpu.sync_copy(r1v, out1.at[pl.ds(base, cb)])
            pltpu.sync_copy(r2v, out2.at[pl.ds(base, cb)])
            return carry

        lax.fori_loop(0, nbc, chunk, 0)

    return kfn


@functools.lru_cache(maxsize=None)
def _make_gather1(ep, n1, d1, chi=CHI):
    rows_w = ep // NW
    cb = min(512, rows_w)
    nbc = rows_w // cb
    grp = cb // chi
    nch = rows_w // chi

    @functools.partial(
        pl.kernel, mesh=_mesh(),
        compiler_params=pltpu.CompilerParams(use_tc_tiling_on_sc=False, needs_layout_passes=False),
        name=f"sc_gather1_{ep}",
        out_type=jax.ShapeDtypeStruct((ep, d1), jnp.float32),
        scratch_types=[pltpu.VMEM((grp, chi), jnp.int32),
                       pltpu.VMEM((cb, d1), jnp.float32),
                       pltpu.SemaphoreType.DMA],
    )
    def kfn(tab1, idx1, out1, i1v, r1v, sem):
        wid = _wid()

        def chunk(j, carry):
            base = wid * rows_w + j * cb
            pltpu.sync_copy(idx1.at[pl.ds(wid * nch + j * grp, grp)], i1v)
            cps = []
            for k in range(grp):
                cps.append(pltpu.async_copy(
                    tab1.at[i1v.at[k]],
                    r1v.at[pl.ds(k * chi, chi)], sem))
            for c in cps:
                c.wait()
            pltpu.sync_copy(r1v, out1.at[pl.ds(base, cb)])
            return carry

        lax.fori_loop(0, nbc, chunk, 0)

    return kfn


def _sc_gather_rows(tab, idx, ep, chi=CHI):
    idx_p = _pad1(idx.astype(jnp.int32), ep, 0)
    return _make_gather1(ep, tab.shape[0], tab.shape[1], chi)(
        tab, idx_p.reshape(ep // chi, chi))


@functools.lru_cache(maxsize=None)
def _make_scatter(ep, npad):
    rows_w = ep // NW
    cb = 512
    nbc = rows_w // cb
    grp = cb // CHI
    nch = rows_w // CHI
    rows_t = npad // 16
    nsp = 2 if rows_t > 128 else 1
    tb = rows_t // nsp

    @functools.partial(
        pl.kernel, mesh=_mesh(),
        compiler_params=pltpu.CompilerParams(use_tc_tiling_on_sc=False, needs_layout_passes=False),
        name=f"sc_scatter_{npad}",
        out_type=jax.ShapeDtypeStruct((2, npad, D), jnp.float32),
        scratch_types=[pltpu.VMEM_SHARED((npad, D), jnp.float32),
                       pltpu.VMEM((grp, CHI), jnp.int32),
                       pltpu.VMEM((cb, D), jnp.float32),
                       pltpu.VMEM((tb, D), jnp.float32)],
    )
    def kfn(msg, dstg, zeros, out, acc, idxv, msgv, tbuf):
        cid = lax.axis_index("c")
        sid = lax.axis_index("s")
        wid = sid * 2 + cid
        for q in range(nsp):
            pltpu.sync_copy(zeros.at[pl.ds(sid * rows_t + q * tb, tb)], tbuf)
            pltpu.sync_copy(tbuf, acc.at[pl.ds(sid * rows_t + q * tb, tb)])
        plsc.subcore_barrier()

        def chunk(j, carry):
            base = wid * rows_w + j * cb
            pltpu.sync_copy(dstg.at[pl.ds(wid * nch + j * grp, grp)], idxv)
            pltpu.sync_copy(msg.at[pl.ds(base, cb)], msgv)
            for k in range(grp):
                pltpu.sync_copy(msgv.at[pl.ds(k * CHI, CHI)],
                                acc.at[idxv.at[k]], add=True)
            return carry

        lax.fori_loop(0, nbc, chunk, 0)
        plsc.subcore_barrier()
        for q in range(nsp):
            pltpu.sync_copy(acc.at[pl.ds(sid * rows_t + q * tb, tb)], tbuf)
            pltpu.sync_copy(
                tbuf, out.at[cid].at[pl.ds(sid * rows_t + q * tb, tb)])

    return kfn


@functools.lru_cache(maxsize=None)
def _make_rankmap(ep, npr, e_real):
    rows_w = ep // NW
    cbe = 2048
    nbc = rows_w // cbe

    @functools.partial(
        pl.kernel, mesh=_mesh(),
        compiler_params=pltpu.CompilerParams(use_tc_tiling_on_sc=False, needs_layout_passes=False),
        name="sc_rankmap",
        out_type=[jax.ShapeDtypeStruct((ep,), jnp.int32),
                  jax.ShapeDtypeStruct((ep,), jnp.int32),
                  jax.ShapeDtypeStruct((ep,), jnp.float32),
                  jax.ShapeDtypeStruct((npr,), jnp.int32)],
        scratch_types=[pltpu.VMEM((npr,), jnp.int32),
                       pltpu.VMEM((1024,), jnp.int32),
                       pltpu.VMEM((cbe,), jnp.int32),
                       pltpu.VMEM((cbe,), jnp.int32),
                       pltpu.VMEM((cbe,), jnp.int32),
                       pltpu.VMEM((cbe,), jnp.int32),
                       pltpu.VMEM((cbe,), jnp.float32)],
    )
    def kfn(mi_hbm, src_hbm, dst_hbm, o_src, o_dst, o_val, o_rank,
            rank_v, mi_v, sbuf, dbuf, eob, dob, vbuf):
        wid = _wid()
        pltpu.sync_copy(mi_hbm, mi_v)
        ksplat = jnp.full((16,), K, jnp.int32)

        def initf(i, carry):
            rank_v[pl.ds(i * 16, 16)] = ksplat
            return carry

        lax.fori_loop(0, npr // 16, initf, 0)

        def setf(j, carry):
            idx16 = mi_v[pl.ds(j * 16, 16)]
            val16 = j * 16 + lax.iota(jnp.int32, 16)
            plsc.store_scatter(rank_v, [idx16], val16, mask=val16 < K)
            return carry

        lax.fori_loop(0, 1024 // 16, setf, 0)

        @pl.when(wid == 0)
        def _():
            pltpu.sync_copy(rank_v, o_rank)

        def chunk(j, carry):
            base = wid * rows_w + j * cbe
            pltpu.sync_copy(src_hbm.at[pl.ds(base, cbe)], sbuf)
            pltpu.sync_copy(dst_hbm.at[pl.ds(base, cbe)], dbuf)

            def grpf(g, c2):
                s16 = sbuf[pl.ds(g * 16, 16)]
                d16 = dbuf[pl.ds(g * 16, 16)]
                sr = plsc.load_gather(rank_v, [s16])
                dr = plsc.load_gather(rank_v, [d16])
                gpos = base + g * 16 + lax.iota(jnp.int32, 16)
                ok = (sr < K) & (dr < K) & (gpos < e_real)
                snt = K + (gpos & 1023)
                eob[pl.ds(g * 16, 16)] = jnp.where(ok, sr, snt)
                dob[pl.ds(g * 16, 16)] = jnp.where(ok, dr, snt)
                vbuf[pl.ds(g * 16, 16)] = jnp.where(
                    ok, jnp.float32(1.0), jnp.float32(0.0))
                return c2

            lax.fori_loop(0, cbe // 16, grpf, 0)
            pltpu.sync_copy(eob, o_src.at[pl.ds(base, cbe)])
            pltpu.sync_copy(dob, o_dst.at[pl.ds(base, cbe)])
            pltpu.sync_copy(vbuf, o_val.at[pl.ds(base, cbe)])
            return carry

        lax.fori_loop(0, nbc, chunk, 0)

    return kfn


def _edge_msg_body(ef_ref, sh_ref, h2s_ref, val_ref, w1_ref, b1_ref, w2_ref,
                   b2_ref, wsh_ref, out_ref, *, prec=None):
    hidden = jnp.maximum(
        jnp.dot(ef_ref[...], w1_ref[...], preferred_element_type=jnp.float32,
                precision=prec)
        + b1_ref[...], 0.0)
    w = jnp.dot(hidden, w2_ref[...], preferred_element_type=jnp.float32,
                precision=prec) + b2_ref[...]
    sv = jnp.dot(sh_ref[...], wsh_ref[...], preferred_element_type=jnp.float32,
                 precision=prec)
    out_ref[...] = (h2s_ref[...] * w + sv) * val_ref[...]


def _edge_msg(g1, g2, valid, p, prec=None):
    ep = g1.shape[0]
    grid = ep // BE
    h2src = g1[:, :D]
    vec = g1[:, D:D + 3] - g2[:, :3]
    r = jnp.linalg.norm(vec, axis=-1, keepdims=True)
    sh = _ref_sh(vec)
    ef = _ref_radial(r)
    return pl.pallas_call(
        functools.partial(_edge_msg_body, prec=prec),
        grid=(grid,),
        in_specs=[
            pl.BlockSpec((BE, 8), lambda i: (i, 0)),
            pl.BlockSpec((BE, 9), lambda i: (i, 0)),
            pl.BlockSpec((BE, D), lambda i: (i, 0)),
            pl.BlockSpec((BE, 1), lambda i: (i, 0)),
            pl.BlockSpec((8, 256), lambda i: (0, 0)),
            pl.BlockSpec((1, 256), lambda i: (0, 0)),
            pl.BlockSpec((256, D), lambda i: (0, 0)),
            pl.BlockSpec((1, D), lambda i: (0, 0)),
            pl.BlockSpec((9, D), lambda i: (0, 0)),
        ],
        out_specs=pl.BlockSpec((BE, D), lambda i: (i, 0)),
        out_shape=jax.ShapeDtypeStruct((ep, D), jnp.float32),
    )(ef, sh, h2src, valid,
      p['W_r1'], p['b_r1'].reshape(1, 256),
      p['W_r2'], p['b_r2'].reshape(1, D),
      p['W_sh'])


def _pad1(x, ep, fill):
    pad = ep - x.shape[0]
    return jnp.concatenate([x, jnp.full((pad,), fill, x.dtype)]) if pad else x


def _conv(h_tab, pos_tab, src_p, dst_p, valid_p, dst_seg, e_len, n_seg, p):
    ep = src_p.shape[0]
    n = h_tab.shape[0]
    h2 = h_tab @ p['W_msg']
    zpad = jnp.zeros((n, 13), jnp.float32)
    tab1 = jnp.concatenate([h2, pos_tab, zpad], axis=1)
    tab2 = jnp.concatenate([pos_tab, zpad], axis=1)
    g1, g2 = _make_gather2(ep, n, 80, n, 16)(
        tab1, src_p.reshape(ep // CHI, CHI), tab2,
        dst_p.reshape(ep // CHI, CHI))
    msg = _edge_msg(g1, g2, valid_p[:, None], p)
    agg = jax.ops.segment_sum(msg[:e_len], dst_seg, num_segments=n_seg)
    return agg * jax.nn.sigmoid(agg @ p['W_gate'] + p['b_gate'])


def _hmp_layer(h, pos, src_p, dst_p, ones_p, dst_u, e_real, p):
    n = h.shape[0]
    ep = src_p.shape[0]
    h_update = _conv(h, pos, src_p, dst_p, ones_p, dst_u, e_real, n, p)
    h_local = h_update + h
    h_scalar = h_local[:, :S]
    score = (jax.nn.relu(h_scalar @ p['W_ms1'] + p['b_ms1']) @ p['W_ms2']
             + p['b_ms2'])[:, 0]
    m = jax.nn.sigmoid(score)
    _, master_idx = jax.lax.top_k(score, K)
    npr = 10240
    mi_p = _pad1(master_idx.astype(jnp.int32), 1024, 0)
    e_srcE, e_dstE, e_valE, rank = _make_rankmap(ep, npr, e_real)(
        mi_p, src_p, dst_p)
    hp_tab = jnp.concatenate(
        [h_local, pos, jnp.zeros((n, 13), jnp.float32)], axis=1)
    hp_m = _sc_gather_rows(hp_tab, master_idx, 1024, chi=32)
    h_master = hp_m[:K, :D]
    pos_master = hp_m[:K, D:D + 3]
    hs = h_master[:, :S]
    logits = (hs @ p['Wq']) @ (hs @ p['Wk']).T / np.sqrt(S)
    attn = jax.nn.softmax(logits, axis=-1)
    _, vcols = jax.lax.top_k(attn, T)
    v_src = jnp.repeat(jnp.arange(K, dtype=jnp.int32), T)
    v_dst = vcols.reshape(-1).astype(jnp.int32)
    v_ok = (v_src != v_dst).astype(jnp.float32)
    npm = 2048
    nel = e_real + 2 * K * T
    epm = (nel + EP_ALIGN - 1) // EP_ALIGN * EP_ALIGN
    spread = K + (jnp.arange(epm - nel, dtype=jnp.int32) & 1023)
    e_src = jnp.concatenate([e_srcE[:e_real], v_src, v_dst, spread])
    e_dst = jnp.concatenate([e_dstE[:e_real], v_dst, v_src, spread])
    e_val = _pad1(
        jnp.concatenate([e_valE[:e_real], v_ok, v_ok]), epm, 0.0)
    e_dst_seg = jnp.concatenate([
        jnp.where(e_valE[:e_real] > 0, e_dstE[:e_real], K), v_dst, v_src])
    hm_pad = jnp.concatenate(
        [h_master, jnp.zeros((npm - K, D), dtype=h_master.dtype)], axis=0)
    pm_pad = jnp.concatenate(
        [pos_master, jnp.zeros((npm - K, 3), dtype=pos_master.dtype)], axis=0)
    hm_update = _conv(hm_pad, pm_pad, e_src, e_dst, e_val,
                      e_dst_seg, nel, K + 1, p)[:K]
    h_hier = hm_update + h_master
    hh_tab = jnp.concatenate([h_hier, jnp.zeros((1, D), h_hier.dtype)])
    h_hier_exp = _sc_gather_rows(hh_tab, rank, 10240, chi=64)[:n]
    m_exp = m[:, None]
    return (1.0 - m_exp) * h_local + m_exp * h_hier_exp


def kernel(atoms, pos, edge_index, batch, params):
    h = params['emb'][atoms]
    src, dst = edge_index[0], edge_index[1]
    e = src.shape[0]
    ep = (e + EP_ALIGN - 1) // EP_ALIGN * EP_ALIGN
    src_p = _pad1(src, ep, 0)
    dst_p = _pad1(dst, ep, 0)
    ones_p = _pad1(jnp.ones((e,), jnp.float32), ep, 0.0)
    stacked = jax.tree.map(lambda *xs: jnp.stack(xs), *params['layers'])

    def body(hc, lp):
        return _hmp_layer(hc, pos, src_p, dst_p, ones_p, dst, e, lp), None

    h, _ = jax.lax.scan(body, h, stacked)
    pooled = jax.ops.segment_sum(h[:, :D], batch, num_segments=NG)
    hidden = jax.nn.relu(pooled @ params['W_p1'] + params['b_p1'])
    return hidden @ params['W_p2'] + params['b_p2']
